# baseline (device time: 45203 ns/iter reference)
import jax
import jax.numpy as jnp
from jax import lax
from jax.experimental import pallas as pl
from jax.experimental.pallas import tpu as pltpu

N_DEV = 8
B = 2
S_Q = 256
S_KV = 256
HQ = 4
DH = 64
D_MODEL = 512
BH = B * HQ
HALO = 128
NGL = 32
NKC = S_KV + 2 * HALO + NGL


def kernel(x, Wq, K_ext, V_ext, Wo):
    K_t = K_ext.transpose(0, 2, 1, 3).reshape(BH, S_KV, DH)
    V_t = V_ext.transpose(0, 2, 1, 3).reshape(BH, S_KV, DH)
    KV = jnp.concatenate([K_t, V_t], axis=0)

    def body(x_ref, wq_ref, kv_ref, wo_ref, out_ref,
             kv_c, q0_buf, partial_send, partial_recv,
             halo_send_sems, halo_recv_sems, gl_send_sems, q0_send_sems,
             partial_send_sem, gl_recv_sem, q0_recv_sem, partial_recv_sems,
             gate_sem):
        my = lax.axis_index("i")
        lp = lax.rem(my + N_DEV - 1, N_DEV)
        rp = lax.rem(my + 1, N_DEV)

        barrier = pltpu.get_barrier_semaphore()
        for nbr in (lp, rp):
            pl.semaphore_signal(barrier, inc=1, device_id=(nbr,),
                                device_id_type=pl.DeviceIdType.MESH)
        pl.semaphore_wait(barrier, 2)

        @pl.when(my != 0)
        def _():
            pl.semaphore_signal(gate_sem, inc=1, device_id=(0,),
                                device_id_type=pl.DeviceIdType.MESH)

        halo_a = pltpu.make_async_remote_copy(
            src_ref=kv_ref.at[:, pl.ds(S_KV - HALO, HALO), :],
            dst_ref=kv_c.at[:, pl.ds(S_KV, HALO), :],
            send_sem=halo_send_sems.at[0], recv_sem=halo_recv_sems.at[0],
            device_id=(rp,), device_id_type=pl.DeviceIdType.MESH)
        halo_b = pltpu.make_async_remote_copy(
            src_ref=kv_ref.at[:, pl.ds(0, HALO), :],
            dst_ref=kv_c.at[:, pl.ds(S_KV + HALO, HALO), :],
            send_sem=halo_send_sems.at[1], recv_sem=halo_recv_sems.at[1],
            device_id=(lp,), device_id_type=pl.DeviceIdType.MESH)
        halo_a.start()
        halo_b.start()
        kv_c[:, 0:S_KV, :] = kv_ref[...]

        q_all = [jnp.dot(x_ref[b], wq_ref[...],
                         preferred_element_type=jnp.float32) * 0.125
                 for b in range(B)]

        gl_descs = [pltpu.make_async_remote_copy(
            src_ref=kv_ref.at[:, pl.ds(0, NGL), :],
            dst_ref=kv_c.at[:, pl.ds(S_KV + 2 * HALO, NGL), :],
            send_sem=gl_send_sems.at[s - 1], recv_sem=gl_recv_sem,
            device_id=(s,), device_id_type=pl.DeviceIdType.MESH)
            for s in range(1, N_DEV)]
        q0_descs = [pltpu.make_async_remote_copy(
            src_ref=q0_buf, dst_ref=q0_buf,
            send_sem=q0_send_sems.at[s - 1], recv_sem=q0_recv_sem,
            device_id=(s,), device_id_type=pl.DeviceIdType.MESH)
            for s in range(1, N_DEV)]

        @pl.when(my == 0)
        def _():
            kv_c[:, S_KV + 2 * HALO:NKC, :] = kv_ref[:, 0:NGL, :]
            for b in range(B):
                q0_buf[b] = q_all[b][0:NGL, :]
            pl.semaphore_wait(gate_sem, N_DEV - 1)
            for d in gl_descs:
                d.start()
            for d in q0_descs:
                d.start()

        @pl.when(my != 0)
        def _():
            q0_descs[0].wait_recv()

        pctx_l, m_l, l_l = [], [], []
        for b in range(B):
            q0b = q0_buf[b]
            for h in range(HQ):
                bh = b * HQ + h
                q0h = q0b[:, h * DH:(h + 1) * DH]
                sp = lax.dot_general(q0h, kv_ref[bh], (((1,), (1,)), ((), ())),
                                     preferred_element_type=jnp.float32)
                mm = jnp.max(sp, axis=1, keepdims=True)
                e = jnp.exp(sp - mm)
                ll = jnp.sum(e, axis=1, keepdims=True)
                pctx = jnp.dot(e, kv_ref[BH + bh],
                               preferred_element_type=jnp.float32)
                pctx_l.append(pctx)
                m_l.append(mm)
                l_l.append(ll)
        own_pctx = jnp.stack(pctx_l)
        own_m = jnp.stack(m_l)
        own_l = jnp.stack(l_l)

        partial_desc = pltpu.make_async_remote_copy(
            src_ref=partial_send, dst_ref=partial_recv.at[my - 1],
            send_sem=partial_send_sem, recv_sem=partial_recv_sems.at[my - 1],
            device_id=(0,), device_id_type=pl.DeviceIdType.MESH)

        @pl.when(my != 0)
        def _():
            partial_send[...] = jnp.concatenate(
                [own_pctx, own_m, own_l,
                 jnp.zeros((BH, NGL, 128 - DH - 2), jnp.float32)], axis=-1)
            partial_desc.start()

        halo_a.wait_recv()
        halo_b.wait_recv()

        @pl.when(my != 0)
        def _():
            gl_descs[0].wait_recv()

        qi = my * S_Q + lax.broadcasted_iota(jnp.int32, (S_Q, NKC), 0)
        col = lax.broadcasted_iota(jnp.int32, (S_Q, NKC), 1)
        ki = jnp.where(
            col < S_KV, my * S_KV + col,
            jnp.where(col < S_KV + HALO, lp * S_KV + HALO + (col - S_KV),
                      jnp.where(col < S_KV + 2 * HALO,
                                rp * S_KV + (col - S_KV - HALO),
                                col - (S_KV + 2 * HALO))))
        is_gl = col >= S_KV + 2 * HALO
        local_ok = jnp.abs(qi - ki) <= HALO
        mask_non_gl = local_ok | ((ki < NGL) & (col < S_KV))
        mask = (mask_non_gl & jnp.logical_not(is_gl)) | (is_gl & (my != 0))

        for b in range(B):
            ctx_heads = []
            for h in range(HQ):
                bh = b * HQ + h
                q = q_all[b][:, h * DH:(h + 1) * DH]
                dn = (((1,), (1,)), ((), ()))
                scores = lax.dot_general(q, kv_c[bh], dn,
                                         preferred_element_type=jnp.float32)
                scores = jnp.where(mask, scores, -1e9)
                mrow = jnp.max(scores, axis=1, keepdims=True)
                w = jnp.exp(scores - mrow)
                w = w / jnp.sum(w, axis=1, keepdims=True)
                ctx = jnp.dot(w, kv_c[BH + bh],
                              preferred_element_type=jnp.float32)
                ctx_heads.append(ctx)
            ctx_b = jnp.concatenate(ctx_heads, axis=1)
            out_ref[b] = jnp.dot(ctx_b, wo_ref[...],
                                 preferred_element_type=jnp.float32)

        recv_descs = [pltpu.make_async_remote_copy(
            src_ref=partial_send, dst_ref=partial_recv.at[s],
            send_sem=partial_send_sem, recv_sem=partial_recv_sems.at[s],
            device_id=(s + 1,), device_id_type=pl.DeviceIdType.MESH)
            for s in range(N_DEV - 1)]

        @pl.when(my == 0)
        def _():
            for d in recv_descs:
                d.wait_recv()
            M = own_m
            for s in range(N_DEV - 1):
                M = jnp.maximum(M, partial_recv[s][:, :, DH:DH + 1])
            num = jnp.exp(own_m - M) * own_pctx
            den = jnp.exp(own_m - M) * own_l
            for s in range(N_DEV - 1):
                pv = partial_recv[s]
                ms = pv[:, :, DH:DH + 1]
                sc = jnp.exp(ms - M)
                num = num + sc * pv[:, :, 0:DH]
                den = den + sc * pv[:, :, DH + 1:DH + 2]
            ctx32 = num / den
            for b in range(B):
                ctx32_b = jnp.concatenate(
                    [ctx32[b * HQ + h] for h in range(HQ)], axis=1)
                out_ref[b, 0:NGL, :] = jnp.dot(
                    ctx32_b, wo_ref[...], preferred_element_type=jnp.float32)

        halo_a.wait_send()
        halo_b.wait_send()

        @pl.when(my == 0)
        def _():
            for d in gl_descs:
                d.wait_send()
            for d in q0_descs:
                d.wait_send()

        @pl.when(my != 0)
        def _():
            partial_desc.wait_send()

    return pl.pallas_call(
        body,
        out_shape=jax.ShapeDtypeStruct((B, S_Q, D_MODEL), jnp.float32),
        in_specs=[pl.BlockSpec(memory_space=pltpu.VMEM)] * 4,
        out_specs=pl.BlockSpec(memory_space=pltpu.VMEM),
        scratch_shapes=[
            pltpu.VMEM((2 * BH, NKC, DH), jnp.float32),
            pltpu.VMEM((B, NGL, HQ * DH), jnp.float32),
            pltpu.VMEM((BH, NGL, 128), jnp.float32),
            pltpu.VMEM((N_DEV - 1, BH, NGL, 128), jnp.float32),
            pltpu.SemaphoreType.DMA((2,)),
            pltpu.SemaphoreType.DMA((2,)),
            pltpu.SemaphoreType.DMA((N_DEV - 1,)),
            pltpu.SemaphoreType.DMA((N_DEV - 1,)),
            pltpu.SemaphoreType.DMA,
            pltpu.SemaphoreType.DMA,
            pltpu.SemaphoreType.DMA,
            pltpu.SemaphoreType.DMA((N_DEV - 1,)),
            pltpu.SemaphoreType.REGULAR,
        ],
        compiler_params=pltpu.CompilerParams(collective_id=0),
    )(x, Wq, KV, Wo)


# device time: 28540 ns/iter; 1.5838x vs baseline; 1.5838x over previous
import jax
import jax.numpy as jnp
from jax import lax
from jax.experimental import pallas as pl
from jax.experimental.pallas import tpu as pltpu

N_DEV = 8
B = 2
S_Q = 256
S_KV = 256
HQ = 4
DH = 64
D_MODEL = 512
BH = B * HQ
HALO = 128
NGL = 32
NKC = S_KV + 2 * HALO + NGL

F32 = jnp.float32
BF16 = jnp.bfloat16


def kernel(x, Wq, K_ext, V_ext, Wo):
    K_t = K_ext.transpose(0, 2, 1, 3).reshape(BH, S_KV, DH)
    V_t = V_ext.transpose(0, 2, 1, 3).reshape(BH, S_KV, DH)
    KV = jnp.concatenate([K_t, V_t], axis=0).astype(BF16)
    x_bf = x.astype(BF16)
    wq_bf = Wq.astype(BF16)
    wo_bf = Wo.astype(BF16)

    def body(x_ref, wq_ref, kv_ref, wo_ref, out_ref,
             kv_c, q0_buf, partial_send, partial_recv,
             halo_send_sems, halo_recv_sems, gl_send_sems, q0_send_sems,
             partial_send_sem, gl_recv_sem, q0_recv_sem, partial_recv_sems,
             gate_sem):
        my = lax.axis_index("i")
        lp = lax.rem(my + N_DEV - 1, N_DEV)
        rp = lax.rem(my + 1, N_DEV)

        @pl.when(my != 0)
        def _():
            pl.semaphore_signal(gate_sem, inc=1, device_id=(0,),
                                device_id_type=pl.DeviceIdType.MESH)

        barrier = pltpu.get_barrier_semaphore()
        for nbr in (lp, rp):
            pl.semaphore_signal(barrier, inc=1, device_id=(nbr,),
                                device_id_type=pl.DeviceIdType.MESH)
        pl.semaphore_wait(barrier, 2)

        halo_a = pltpu.make_async_remote_copy(
            src_ref=kv_ref.at[:, pl.ds(S_KV - HALO, HALO), :],
            dst_ref=kv_c.at[:, pl.ds(S_KV, HALO), :],
            send_sem=halo_send_sems.at[0], recv_sem=halo_recv_sems.at[0],
            device_id=(rp,), device_id_type=pl.DeviceIdType.MESH)
        halo_b = pltpu.make_async_remote_copy(
            src_ref=kv_ref.at[:, pl.ds(0, HALO), :],
            dst_ref=kv_c.at[:, pl.ds(S_KV + HALO, HALO), :],
            send_sem=halo_send_sems.at[1], recv_sem=halo_recv_sems.at[1],
            device_id=(lp,), device_id_type=pl.DeviceIdType.MESH)
        halo_a.start()
        halo_b.start()

        gl_descs = [pltpu.make_async_remote_copy(
            src_ref=kv_ref.at[:, pl.ds(0, NGL), :],
            dst_ref=kv_c.at[:, pl.ds(S_KV + 2 * HALO, NGL), :],
            send_sem=gl_send_sems.at[s - 1], recv_sem=gl_recv_sem,
            device_id=(s,), device_id_type=pl.DeviceIdType.MESH)
            for s in range(1, N_DEV)]
        q0_descs = [pltpu.make_async_remote_copy(
            src_ref=q0_buf, dst_ref=q0_buf,
            send_sem=q0_send_sems.at[s - 1], recv_sem=q0_recv_sem,
            device_id=(s,), device_id_type=pl.DeviceIdType.MESH)
            for s in range(1, N_DEV)]

        @pl.when(my == 0)
        def _():
            for b in range(B):
                q0_buf[b] = jnp.dot(x_ref[b, 0:NGL, :], wq_ref[...],
                                    preferred_element_type=F32) * 0.125
            kv_c[:, S_KV + 2 * HALO:NKC, :] = kv_ref[:, 0:NGL, :]
            pl.semaphore_wait(gate_sem, N_DEV - 1)
            for d in q0_descs:
                d.start()
            for d in gl_descs:
                d.start()

        kv_c[:, 0:S_KV, :] = kv_ref[...]

        q_all = [(jnp.dot(x_ref[b], wq_ref[...],
                          preferred_element_type=F32) * 0.125).astype(BF16)
                 for b in range(B)]

        @pl.when(my != 0)
        def _():
            q0_descs[0].wait_recv()

        pctx_l, m_l, l_l = [], [], []
        for b in range(B):
            q0b = q0_buf[b].astype(BF16)
            for h in range(HQ):
                bh = b * HQ + h
                q0h = q0b[:, h * DH:(h + 1) * DH]
                sp = lax.dot_general(q0h, kv_ref[bh], (((1,), (1,)), ((), ())),
                                     preferred_element_type=F32)
                mm = jnp.max(sp, axis=1, keepdims=True)
                e = jnp.exp(sp - mm)
                ll = jnp.sum(e, axis=1, keepdims=True)
                pctx = jnp.dot(e.astype(BF16), kv_ref[BH + bh],
                               preferred_element_type=F32)
                pctx_l.append(pctx)
                m_l.append(mm)
                l_l.append(ll)
        own_pctx = jnp.stack(pctx_l)
        own_m = jnp.stack(m_l)
        own_l = jnp.stack(l_l)

        partial_desc = pltpu.make_async_remote_copy(
            src_ref=partial_send, dst_ref=partial_recv.at[my - 1],
            send_sem=partial_send_sem, recv_sem=partial_recv_sems.at[my - 1],
            device_id=(0,), device_id_type=pl.DeviceIdType.MESH)

        @pl.when(my != 0)
        def _():
            partial_send[...] = jnp.concatenate(
                [own_pctx, own_m, own_l,
                 jnp.zeros((BH, NGL, 128 - DH - 2), F32)], axis=-1)
            partial_desc.start()

        halo_a.wait_recv()
        halo_b.wait_recv()

        @pl.when(my != 0)
        def _():
            gl_descs[0].wait_recv()

        qi = my * S_Q + lax.broadcasted_iota(jnp.int32, (S_Q, NKC), 0)
        col = lax.broadcasted_iota(jnp.int32, (S_Q, NKC), 1)
        ki = jnp.where(
            col < S_KV, my * S_KV + col,
            jnp.where(col < S_KV + HALO, lp * S_KV + HALO + (col - S_KV),
                      jnp.where(col < S_KV + 2 * HALO,
                                rp * S_KV + (col - S_KV - HALO),
                                col - (S_KV + 2 * HALO))))
        is_gl = col >= S_KV + 2 * HALO
        local_ok = jnp.abs(qi - ki) <= HALO
        mask_non_gl = local_ok | ((ki < NGL) & (col < S_KV))
        mask = (mask_non_gl & jnp.logical_not(is_gl)) | (is_gl & (my != 0))

        for b in range(B):
            ctx_heads = []
            for h in range(HQ):
                bh = b * HQ + h
                q = q_all[b][:, h * DH:(h + 1) * DH]
                dn = (((1,), (1,)), ((), ()))
                scores = lax.dot_general(q, kv_c[bh], dn,
                                         preferred_element_type=F32)
                scores = jnp.where(mask, scores, -1e9)
                mrow = jnp.max(scores, axis=1, keepdims=True)
                w = jnp.exp(scores - mrow)
                w = w / jnp.sum(w, axis=1, keepdims=True)
                ctx = jnp.dot(w.astype(BF16), kv_c[BH + bh],
                              preferred_element_type=F32)
                ctx_heads.append(ctx)
            ctx_b = jnp.concatenate(ctx_heads, axis=1).astype(BF16)
            out_ref[b] = jnp.dot(ctx_b, wo_ref[...],
                                 preferred_element_type=F32)

        recv_descs = [pltpu.make_async_remote_copy(
            src_ref=partial_send, dst_ref=partial_recv.at[s],
            send_sem=partial_send_sem, recv_sem=partial_recv_sems.at[s],
            device_id=(s + 1,), device_id_type=pl.DeviceIdType.MESH)
            for s in range(N_DEV - 1)]

        @pl.when(my == 0)
        def _():
            for d in recv_descs:
                d.wait_recv()
            M = own_m
            for s in range(N_DEV - 1):
                M = jnp.maximum(M, partial_recv[s][:, :, DH:DH + 1])
            num = jnp.exp(own_m - M) * own_pctx
            den = jnp.exp(own_m - M) * own_l
            for s in range(N_DEV - 1):
                pv = partial_recv[s]
                ms = pv[:, :, DH:DH + 1]
                sc = jnp.exp(ms - M)
                num = num + sc * pv[:, :, 0:DH]
                den = den + sc * pv[:, :, DH + 1:DH + 2]
            ctx32 = num / den
            for b in range(B):
                ctx32_b = jnp.concatenate(
                    [ctx32[b * HQ + h] for h in range(HQ)],
                    axis=1).astype(BF16)
                out_ref[b, 0:NGL, :] = jnp.dot(
                    ctx32_b, wo_ref[...], preferred_element_type=F32)

        halo_a.wait_send()
        halo_b.wait_send()

        @pl.when(my == 0)
        def _():
            for d in gl_descs:
                d.wait_send()
            for d in q0_descs:
                d.wait_send()

        @pl.when(my != 0)
        def _():
            partial_desc.wait_send()

    return pl.pallas_call(
        body,
        out_shape=jax.ShapeDtypeStruct((B, S_Q, D_MODEL), F32),
        in_specs=[pl.BlockSpec(memory_space=pltpu.VMEM)] * 4,
        out_specs=pl.BlockSpec(memory_space=pltpu.VMEM),
        scratch_shapes=[
            pltpu.VMEM((2 * BH, NKC, DH), BF16),
            pltpu.VMEM((B, NGL, HQ * DH), F32),
            pltpu.VMEM((BH, NGL, 128), F32),
            pltpu.VMEM((N_DEV - 1, BH, NGL, 128), F32),
            pltpu.SemaphoreType.DMA((2,)),
            pltpu.SemaphoreType.DMA((2,)),
            pltpu.SemaphoreType.DMA((N_DEV - 1,)),
            pltpu.SemaphoreType.DMA((N_DEV - 1,)),
            pltpu.SemaphoreType.DMA,
            pltpu.SemaphoreType.DMA,
            pltpu.SemaphoreType.DMA,
            pltpu.SemaphoreType.DMA((N_DEV - 1,)),
            pltpu.SemaphoreType.REGULAR,
        ],
        compiler_params=pltpu.CompilerParams(collective_id=0),
    )(x_bf, wq_bf, KV, wo_bf)
